# baseline (speedup 1.0000x reference)
import dataclasses

import jax
import jax.numpy as jnp
from jax import lax
from jax.experimental import pallas as pl
from jax.experimental.pallas import tpu as pltpu
from jax.experimental.pallas import tpu_sc as plsc

B = 16384
K = 32
L = 1000000
GROUPS = 128 // K
NC = 2
NS = 16
NW = NC * NS
B_PER_W = B // NW

TCHUNK = 8192
TGRID = -(-L // TCHUNK)
WROWS = TGRID * (TCHUNK // GROUPS)


def _widen_body(i_ref, o_ref):
    v = i_ref[...]
    nsub = TCHUNK // 512
    nx = 11 * nsub // 16
    split = nx * 512
    vt1 = v[:, :split].T
    vt2 = lax.dot_general(
        v[:, split:],
        jnp.eye(K, dtype=jnp.float32),
        (((0,), (0,)), ((), ())),
        precision=lax.Precision.HIGHEST,
    )
    for s in range(nsub):
        vt = vt1 if s < nx else vt2
        base = s * 512 - (0 if s < nx else split)
        for m in range(4):
            o_ref[pl.ds(s * 128, 128), pl.ds(32 * m, 32)] = vt[
                base + 128 * m : base + 128 * (m + 1), :
            ]


def _tc_widen(table_t):
    return pl.pallas_call(
        _widen_body,
        grid=(TGRID,),
        in_specs=[pl.BlockSpec((K, TCHUNK), lambda i: (0, i))],
        out_specs=pl.BlockSpec((TCHUNK // GROUPS, 128), lambda i: (i, 0)),
        out_shape=jax.ShapeDtypeStruct((WROWS, 128), jnp.float32),
        compiler_params=pltpu.CompilerParams(
            dimension_semantics=("parallel",),
        ),
    )(table_t)


def _sc_gather_wide(x_wide, table_wide):
    mesh = plsc.VectorSubcoreMesh(core_axis_name="c", subcore_axis_name="s")
    cp = pltpu.CompilerParams()
    if "needs_layout_passes" in pltpu.CompilerParams.__dataclass_fields__:
        cp = dataclasses.replace(cp, needs_layout_passes=False)

    @pl.kernel(
        mesh=mesh,
        compiler_params=cp,
        out_type=jax.ShapeDtypeStruct((B, 128), jnp.float32),
        scratch_types=[
            pltpu.VMEM((B_PER_W,), jnp.int32),
            pltpu.VMEM((B_PER_W, 128), jnp.float32),
            pltpu.SemaphoreType.DMA,
        ],
    )
    def gather_kernel(table_hbm, idx_hbm, out_hbm, idx_v, rows_v, sem):
        wid = lax.axis_index("s") * NC + lax.axis_index("c")
        base = wid * B_PER_W
        pltpu.sync_copy(idx_hbm.at[pl.ds(base, B_PER_W)], idx_v)
        pltpu.async_copy(table_hbm.at[idx_v], rows_v, sem).wait()
        pltpu.sync_copy(rows_v, out_hbm.at[pl.ds(base, B_PER_W)])

    return gather_kernel(table_wide, x_wide)


def _normalize_body(w_ref, g_ref, o_ref):
    w = w_ref[...]
    g = g_ref[...]
    lane_group = lax.broadcasted_iota(jnp.int32, (B, 128), 1) // K
    a = jnp.where(g == lane_group, w, 0.0)
    raw = a[:, 0:32] + a[:, 32:64] + a[:, 64:96] + a[:, 96:128]
    ss = jnp.sum(raw * raw, axis=0)
    scale = lax.rsqrt(ss * (1.0 / B))
    o_ref[...] = raw * scale[None, :]


def _tc_normalize(wide_rows, group):
    return pl.pallas_call(
        _normalize_body,
        out_shape=jax.ShapeDtypeStruct((B, K), jnp.float32),
    )(wide_rows, group)


def kernel(x, eigenfuncs):
    x = x.astype(jnp.int32)
    table_wide = _tc_widen(eigenfuncs.T)
    wide_idx = ((x >> 9) << 7) + (x & 127)
    group = (x >> 7) & 3
    wide_rows = _sc_gather_wide(wide_idx, table_wide)
    return _tc_normalize(wide_rows, group.reshape(B, 1))


# speedup vs baseline: 1.0504x; 1.0504x over previous
import dataclasses

import jax
import jax.numpy as jnp
from jax import lax
from jax.experimental import pallas as pl
from jax.experimental.pallas import tpu as pltpu
from jax.experimental.pallas import tpu_sc as plsc

B = 16384
K = 32
L = 1000000
GROUPS = 128 // K
NC = 2
NS = 16
NW = NC * NS
B_PER_W = B // NW

TCHUNK = 32768
TGRID = -(-L // TCHUNK)
WROWS = TGRID * (TCHUNK // GROUPS)


def _widen_body(i_ref, o_ref):
    vt = i_ref[...].T
    for s in range(TCHUNK // 512):
        base = s * 512
        for m in range(4):
            o_ref[pl.ds(s * 128, 128), pl.ds(32 * m, 32)] = vt[
                base + 128 * m : base + 128 * (m + 1), :
            ]


def _tc_widen(table_t):
    return pl.pallas_call(
        _widen_body,
        grid=(TGRID,),
        in_specs=[pl.BlockSpec((K, TCHUNK), lambda i: (0, i))],
        out_specs=pl.BlockSpec((TCHUNK // GROUPS, 128), lambda i: (i, 0)),
        out_shape=jax.ShapeDtypeStruct((WROWS, 128), jnp.float32),
        compiler_params=pltpu.CompilerParams(
            dimension_semantics=("parallel",),
        ),
    )(table_t)


def _sc_gather_wide(x_wide, table_wide):
    mesh = plsc.VectorSubcoreMesh(core_axis_name="c", subcore_axis_name="s")
    cp = pltpu.CompilerParams()
    if "needs_layout_passes" in pltpu.CompilerParams.__dataclass_fields__:
        cp = dataclasses.replace(cp, needs_layout_passes=False)

    @pl.kernel(
        mesh=mesh,
        compiler_params=cp,
        out_type=jax.ShapeDtypeStruct((B, 128), jnp.float32),
        scratch_types=[
            pltpu.VMEM((B_PER_W,), jnp.int32),
            pltpu.VMEM((B_PER_W, 128), jnp.float32),
            pltpu.SemaphoreType.DMA,
        ],
    )
    def gather_kernel(table_hbm, idx_hbm, out_hbm, idx_v, rows_v, sem):
        wid = lax.axis_index("s") * NC + lax.axis_index("c")
        base = wid * B_PER_W
        pltpu.sync_copy(idx_hbm.at[pl.ds(base, B_PER_W)], idx_v)
        pltpu.async_copy(table_hbm.at[idx_v], rows_v, sem).wait()
        pltpu.sync_copy(rows_v, out_hbm.at[pl.ds(base, B_PER_W)])

    return gather_kernel(table_wide, x_wide)


def _normalize_body(w_ref, g_ref, o_ref):
    w = w_ref[...]
    g = g_ref[...]
    lane_group = lax.broadcasted_iota(jnp.int32, (B, 128), 1) // K
    a = jnp.where(g == lane_group, w, 0.0)
    raw = a[:, 0:32] + a[:, 32:64] + a[:, 64:96] + a[:, 96:128]
    ss = jnp.sum(raw * raw, axis=0)
    scale = lax.rsqrt(ss * (1.0 / B))
    o_ref[...] = raw * scale[None, :]


def _tc_normalize(wide_rows, group):
    return pl.pallas_call(
        _normalize_body,
        out_shape=jax.ShapeDtypeStruct((B, K), jnp.float32),
    )(wide_rows, group)


def kernel(x, eigenfuncs):
    x = x.astype(jnp.int32)
    table_wide = _tc_widen(eigenfuncs.T)
    wide_idx = ((x >> 9) << 7) + (x & 127)
    group = (x >> 7) & 3
    wide_rows = _sc_gather_wide(wide_idx, table_wide)
    return _tc_normalize(wide_rows, group.reshape(B, 1))
